# Initial kernel scaffold; baseline (speedup 1.0000x reference)
import jax
import jax.numpy as jnp
from jax.experimental import pallas as pl


def kernel(x, gt, feat, sdf_data, f_W, f_b, s_W, s_b):
    raise NotImplementedError("write your pallas kernel here")


# speedup vs baseline: 1962.8816x; 1962.8816x over previous
import functools

import jax
import jax.numpy as jnp
from jax.experimental import pallas as pl
from jax.experimental.pallas import tpu as pltpu

_FOLD = 8
_NSP = 4
_NCLS = 21
_W = 64
_PAIR = 2
_NB = _NSP * _PAIR
_NPIX = _W * _W * _PAIR
_NPAIRS = _FOLD * _FOLD // _PAIR


def _tile_kernel(x_ref, gt_ref, feat_ref, sdf_ref, fW_ref, fb_ref, sW_ref,
                 sb_ref, cent_ref, lab_ref, map_ref):
    f32 = jnp.float32
    mm = functools.partial(jax.lax.dot_general, preferred_element_type=f32)
    cP = functools.partial(mm, dimension_numbers=(((1,), (1,)), ((), ())))

    xf = x_ref[0].reshape(x_ref.shape[1], _NPIX)
    pix = feat_ref[0].reshape(feat_ref.shape[1], _NPIX)
    sdfx = sdf_ref[0].reshape(sdf_ref.shape[1], _NPIX)

    deep = mm(fW_ref[...], xf, (((0,), (0,)), ((), ()))) + fb_ref[...]
    sdfp = mm(sW_ref[...], sdfx, (((0,), (0,)), ((), ()))) + sb_ref[...]

    p = jax.lax.broadcasted_iota(jnp.int32, (1, _NPIX), 1)
    sub = (p // _W) % _PAIR
    q = 2 * (p // (_NPIX // 2)) + (p % _W) // 32
    lab = _NSP * sub + q

    s8 = jax.lax.broadcasted_iota(jnp.int32, (_NB, _NPIX), 0)
    same = (s8 // _NSP) == sub
    onehot = jnp.where(s8 == lab, 1.0, 0.0).astype(f32)
    disp = jnp.abs(s8 % _NSP - q)
    w8 = jnp.where(same & (disp == 1), 2.0,
                   jnp.where(same, 1.0, 0.0)).astype(f32)

    inv_cnt = 1.0 / (_W * _W / _NSP)
    spix0 = cP(onehot, pix) * inv_cnt
    deep_c = cP(onehot, deep) * inv_cnt
    sdf_c = cP(onehot, sdfp) * inv_cnt

    pn_pix = jnp.sum(pix * pix, axis=0, keepdims=True)
    pn_deep = jnp.sum(deep * deep, axis=0, keepdims=True)
    pn_sdf = jnp.sum(sdfp * sdfp, axis=0, keepdims=True)

    def d8_of(cT, arr, pn):
        sn = jnp.sum(cT * cT, axis=1, keepdims=True)
        cross = mm(cT, arr, (((1,), (0,)), ((), ())))
        return pn + sn - 2.0 * cross

    d_fix = d8_of(deep_c, deep, pn_deep) + d8_of(sdf_c, sdfp, pn_sdf)

    spixT = spix0
    aff = None
    for k in range(2):
        d8 = d8_of(spixT, pix, pn_pix) + d_fix
        dm = jnp.where(same, d8, 1e16)
        mn = jnp.min(dm, axis=0, keepdims=True)
        e = w8 * jnp.exp(mn - dm)
        aff = e / jnp.sum(e, axis=0, keepdims=True)
        if k == 0:
            sp_new = cP(aff, pix)
            spixT = sp_new / (jnp.sum(aff, axis=1, keepdims=True) + 1e-16)

    mx = jnp.max(aff, axis=0, keepdims=True)
    cand = jnp.where(aff == mx, s8, _NB)
    idx = jnp.min(cand, axis=0, keepdims=True)
    mask8 = jnp.where(s8 == idx, 1.0, 0.0).astype(f32)
    sim = aff * mask8

    cent_ref[0, 0] = (cP(sim, pix) + spix0) / (
        jnp.sum(sim, axis=1, keepdims=True) + 1.0)

    g = gt_ref[0, 0]
    c_iota = jax.lax.broadcasted_iota(jnp.int32, (_NCLS, _NPIX), 0)
    gh = jnp.where(c_iota == g, 1.0, 0.0).astype(f32)
    lab_ref[0, 0] = cP(gh, mask8)

    fold_base = pl.program_id(1) * _FOLD + pl.program_id(2) * _PAIR
    map_ref[0, 0] = idx.astype(f32) + (fold_base * _NSP).astype(f32)


def kernel(x, gt, feat, sdf_data, f_W, f_b, s_W, s_b):
    B = x.shape[0]
    C = feat.shape[1]
    gt_t = (gt.reshape(B, _FOLD, _W, _FOLD // _PAIR, _PAIR * _W)
              .transpose(0, 1, 3, 2, 4)
              .reshape(B, _NPAIRS, 1, _NPIX).astype(jnp.int32))
    fb2 = f_b.reshape(-1, 1)
    sb2 = s_b.reshape(-1, 1)

    cent, labs, smap = pl.pallas_call(
        _tile_kernel,
        grid=(B, _FOLD, _FOLD // _PAIR),
        in_specs=[
            pl.BlockSpec((1, x.shape[1], _W, _PAIR * _W),
                         lambda b, i, j: (b, 0, i, j)),
            pl.BlockSpec((1, 1, 1, _NPIX),
                         lambda b, i, j: (b, i * (_FOLD // _PAIR) + j, 0, 0)),
            pl.BlockSpec((1, C, _W, _PAIR * _W), lambda b, i, j: (b, 0, i, j)),
            pl.BlockSpec((1, sdf_data.shape[1], _W, _PAIR * _W),
                         lambda b, i, j: (b, 0, i, j)),
            pl.BlockSpec(f_W.shape, lambda b, i, j: (0, 0)),
            pl.BlockSpec((f_b.shape[0], 1), lambda b, i, j: (0, 0)),
            pl.BlockSpec(s_W.shape, lambda b, i, j: (0, 0)),
            pl.BlockSpec((s_b.shape[0], 1), lambda b, i, j: (0, 0)),
        ],
        out_specs=[
            pl.BlockSpec((1, 1, _NB, C),
                         lambda b, i, j: (b, i * (_FOLD // _PAIR) + j, 0, 0)),
            pl.BlockSpec((1, 1, _NCLS, _NB),
                         lambda b, i, j: (b, i * (_FOLD // _PAIR) + j, 0, 0)),
            pl.BlockSpec((1, 1, 1, _NPIX),
                         lambda b, i, j: (b, i * (_FOLD // _PAIR) + j, 0, 0)),
        ],
        out_shape=[
            jax.ShapeDtypeStruct((B, _NPAIRS, _NB, C), jnp.float32),
            jax.ShapeDtypeStruct((B, _NPAIRS, _NCLS, _NB), jnp.float32),
            jax.ShapeDtypeStruct((B, _NPAIRS, 1, _NPIX), jnp.float32),
        ],
        compiler_params=pltpu.CompilerParams(
            dimension_semantics=("parallel", "parallel", "parallel")),
    )(x, gt_t, feat, sdf_data, f_W, fb2, s_W, sb2)

    center_feat = cent.reshape(B, _FOLD * _FOLD * _NSP, C)
    labels = labs.transpose(0, 2, 1, 3).reshape(B, _NCLS, _FOLD * _FOLD * _NSP)
    spix_map = (smap.reshape(B, _FOLD, _FOLD // _PAIR, _W, _PAIR * _W)
                    .transpose(0, 1, 3, 2, 4)
                    .reshape(B, _FOLD * _W, _FOLD * _W))
    return center_feat, labels, spix_map
